# Initial kernel scaffold; baseline (speedup 1.0000x reference)
import jax
import jax.numpy as jnp
from jax.experimental import pallas as pl


def kernel(x, Wg, W1, b1, W2, b2):
    raise NotImplementedError("write your pallas kernel here")


# speedup vs baseline: 21.5235x; 21.5235x over previous
import functools

import jax
import jax.numpy as jnp
from jax import lax
from jax.experimental import pallas as pl
from jax.experimental.pallas import tpu as pltpu
from jax.experimental.pallas import tpu_sc as plsc

N = 4096
D = 768
E = 64
F = 3072
TM = 128
NBLK = N // TM
MAXT = E + N // TM
META_LEN = 128
NPAD = MAXT * TM


def _route_body(x_ref, wg_ref, ppos_ref, meta_ref, oh_ref):
    x = x_ref[...]
    wg = wg_ref[...]
    logits = lax.dot_general(x, wg, (((1,), (1,)), ((), ())),
                             preferred_element_type=jnp.float32)
    mx = jnp.max(logits, axis=1, keepdims=True)
    eiota = lax.broadcasted_iota(jnp.int32, (N, E), 1)
    top1 = jnp.min(jnp.where(logits == mx, eiota, E), axis=1, keepdims=True)
    oh = (eiota == top1).astype(jnp.float32)
    oh_ref[...] = oh

    ones_1n = jnp.ones((1, N), jnp.float32)
    counts = lax.dot_general(ones_1n, oh, (((1,), (0,)), ((), ())), precision=lax.Precision.HIGHEST)
    nt = jnp.floor((counts + (TM - 1.0)) * (1.0 / TM))
    lt_strict = (lax.broadcasted_iota(jnp.int32, (E, E), 0) <
                 lax.broadcasted_iota(jnp.int32, (E, E), 1)).astype(jnp.float32)
    tile_cum = lax.dot_general(nt, lt_strict, (((1,), (0,)), ((), ())), precision=lax.Precision.HIGHEST)
    tile_end = tile_cum + nt
    total_tiles = jnp.sum(nt)
    poffset = TM * tile_cum

    lt_inc = (lax.broadcasted_iota(jnp.int32, (TM, TM), 1) <=
              lax.broadcasted_iota(jnp.int32, (TM, TM), 0)).astype(jnp.float32)
    ones_e1 = jnp.ones((E, 1), jnp.float32)

    carry = jnp.zeros((1, E), jnp.float32)
    for t in range(NBLK):
        oh_blk = oh[t * TM:(t + 1) * TM, :]
        csum_blk = lax.dot_general(lt_inc, oh_blk, (((1,), (0,)), ((), ())), precision=lax.Precision.HIGHEST) + carry
        val = csum_blk - 1.0 + poffset
        ppos_blk = lax.dot_general(oh_blk * val, ones_e1,
                                   (((1,), (0,)), ((), ())), precision=lax.Precision.HIGHEST)
        ppos_ref[t * TM:(t + 1) * TM, :] = ppos_blk.astype(jnp.int32)
        carry = carry + jnp.sum(oh_blk, axis=0, keepdims=True)

    eye = (lax.broadcasted_iota(jnp.int32, (E, E), 0) ==
           lax.broadcasted_iota(jnp.int32, (E, E), 1)).astype(jnp.float32)
    tile_end_col = lax.dot_general(eye, tile_end, (((1,), (1,)), ((), ())), precision=lax.Precision.HIGHEST)
    counts_col = lax.dot_general(eye, counts, (((1,), (1,)), ((), ())), precision=lax.Precision.HIGHEST)
    eiota_col = lax.broadcasted_iota(jnp.int32, (E, 1), 0).astype(jnp.float32)
    last_used = jnp.max(jnp.where(counts_col > 0.0, eiota_col, 0.0))
    tiota = lax.broadcasted_iota(jnp.int32, (1, META_LEN), 1).astype(jnp.float32)
    mask = (tile_end_col <= tiota).astype(jnp.float32)
    ones_1e = jnp.ones((1, E), jnp.float32)
    e_map = lax.dot_general(ones_1e, mask, (((1,), (0,)), ((), ())), precision=lax.Precision.HIGHEST)
    e_map = jnp.minimum(e_map, last_used)
    meta = jnp.where(tiota == float(MAXT), total_tiles, e_map)
    meta_ref[...] = meta.astype(jnp.int32)


def _ffn_body(meta_ref, xs_ref, w1_ref, b1_ref, w2_ref, b2_ref, ys_ref, h_ref):
    t = pl.program_id(0)

    @pl.when(t < meta_ref[MAXT])
    def _():
        xt = xs_ref[...]
        h = jnp.dot(xt, w1_ref[0], preferred_element_type=jnp.float32)
        h = h + b1_ref[0]
        h_ref[...] = 0.5 * h * (1.0 + lax.erf(h * 0.7071067811865476))
        ys = jnp.dot(h_ref[...], w2_ref[0], preferred_element_type=jnp.float32)
        ys_ref[...] = ys + b2_ref[0]


def _route(x_flat, wg):
    ppos2, meta2 = pl.pallas_call(
        _route_body,
        out_shape=[
            jax.ShapeDtypeStruct((N, 1), jnp.int32),
            jax.ShapeDtypeStruct((1, META_LEN), jnp.int32),
        ],
        scratch_shapes=[pltpu.VMEM((N, E), jnp.float32)],
    )(x_flat, wg)
    return ppos2.reshape(N), meta2.reshape(META_LEN)


def _ffn(meta, xs_pad, w1, b1, w2, b2):
    grid_spec = pltpu.PrefetchScalarGridSpec(
        num_scalar_prefetch=1,
        grid=(MAXT,),
        in_specs=[
            pl.BlockSpec((TM, D), lambda t, s: (t, 0)),
            pl.BlockSpec((1, D, F), lambda t, s: (s[t], 0, 0)),
            pl.BlockSpec((1, 1, F), lambda t, s: (s[t], 0, 0)),
            pl.BlockSpec((1, F, D), lambda t, s: (s[t], 0, 0)),
            pl.BlockSpec((1, 1, D), lambda t, s: (s[t], 0, 0)),
        ],
        out_specs=pl.BlockSpec((TM, D), lambda t, s: (t, 0)),
        scratch_shapes=[pltpu.VMEM((TM, F), jnp.float32)],
    )
    return pl.pallas_call(
        _ffn_body,
        grid_spec=grid_spec,
        out_shape=jax.ShapeDtypeStruct((NPAD, D), jnp.float32),
        compiler_params=pltpu.CompilerParams(
            vmem_limit_bytes=120 * 1024 * 1024),
    )(meta, xs_pad, w1, b1.reshape(E, 1, F), w2, b2.reshape(E, 1, D))


def _sc_dispatch(x_flat, ppos):
    info = plsc.get_sparse_core_info()
    nc, ns = info.num_cores, info.num_subcores
    nw = nc * ns
    chunk = N // nw
    mesh = plsc.VectorSubcoreMesh(core_axis_name="c", subcore_axis_name="s")

    @functools.partial(
        pl.kernel, mesh=mesh,
        out_type=jax.ShapeDtypeStruct((NPAD, D), jnp.float32),
        scratch_types=[
            pltpu.VMEM((chunk,), jnp.int32),
            pltpu.VMEM((chunk, D), jnp.float32),
            pltpu.SemaphoreType.DMA,
        ],
    )
    def scatter_k(x_hbm, ppos_hbm, xs_hbm, idx_v, rows_v, sem):
        wid = lax.axis_index("s") * nc + lax.axis_index("c")
        base = wid * chunk
        pltpu.sync_copy(ppos_hbm.at[pl.ds(base, chunk)], idx_v)
        pltpu.sync_copy(x_hbm.at[pl.ds(base, chunk)], rows_v)
        pltpu.async_copy(rows_v, xs_hbm.at[idx_v], sem).wait()

    return scatter_k(x_flat, ppos)


def _sc_combine(ys_pad, ppos):
    info = plsc.get_sparse_core_info()
    nc, ns = info.num_cores, info.num_subcores
    nw = nc * ns
    chunk = N // nw
    mesh = plsc.VectorSubcoreMesh(core_axis_name="c", subcore_axis_name="s")

    @functools.partial(
        pl.kernel, mesh=mesh,
        out_type=jax.ShapeDtypeStruct((N, D), jnp.float32),
        scratch_types=[
            pltpu.VMEM((chunk,), jnp.int32),
            pltpu.VMEM((chunk, D), jnp.float32),
            pltpu.SemaphoreType.DMA,
        ],
    )
    def gather_k(ys_hbm, ppos_hbm, y_hbm, idx_v, rows_v, sem):
        wid = lax.axis_index("s") * nc + lax.axis_index("c")
        base = wid * chunk
        pltpu.sync_copy(ppos_hbm.at[pl.ds(base, chunk)], idx_v)
        pltpu.async_copy(ys_hbm.at[idx_v], rows_v, sem).wait()
        pltpu.sync_copy(rows_v, y_hbm.at[pl.ds(base, chunk)])

    return gather_k(ys_pad, ppos)


def kernel(x, Wg, W1, b1, W2, b2):
    Bb, T, Dm = x.shape
    x_flat = x.reshape(Bb * T, Dm)
    ppos, meta = _route(x_flat, Wg)
    xs_pad = _sc_dispatch(x_flat, ppos)
    ys_pad = _ffn(meta, xs_pad, W1, b1, W2, b2)
    y_flat = _sc_combine(ys_pad, ppos)
    return y_flat.reshape(Bb, T, Dm)
